# Initial kernel scaffold; baseline (speedup 1.0000x reference)
import jax
import jax.numpy as jnp
from jax.experimental import pallas as pl


def kernel(h, edge_index):
    raise NotImplementedError("write your pallas kernel here")


# speedup vs baseline: 3.3010x; 3.3010x over previous
import functools

import jax
import jax.numpy as jnp
from jax import lax
from jax.experimental import pallas as pl
from jax.experimental.pallas import tpu as pltpu
from jax.experimental.pallas import tpu_sc as plsc

E = 320000
D = 128
L = 16
NC = 2
NS = 16
NW = NC * NS
PER_TILE = E // NW
C = 80
N_CHUNKS = PER_TILE // C


@functools.partial(
    pl.kernel,
    mesh=plsc.VectorSubcoreMesh(core_axis_name="c", subcore_axis_name="s"),
    out_type=jax.ShapeDtypeStruct((E,), jnp.float32),
    scratch_types=[
        pltpu.VMEM((C,), jnp.int32),
        pltpu.VMEM((C,), jnp.int32),
        pltpu.VMEM((C, D), jnp.float32),
        pltpu.VMEM((C, D), jnp.float32),
        pltpu.VMEM((L,), jnp.float32),
        pltpu.VMEM((PER_TILE,), jnp.float32),
        pltpu.SemaphoreType.DMA,
    ],
)
def _edge_dot(src_hbm, dst_hbm, h_hbm, out_hbm,
              idxu, idxv, urows, vrows, sbuf, outall, sem):
    wid = lax.axis_index("s") * NC + lax.axis_index("c")
    base = wid * PER_TILE
    lanes = lax.iota(jnp.int32, L)

    def chunk_body(i, carry):
        off = base + i * C
        pltpu.sync_copy(src_hbm.at[pl.ds(off, C)], idxu)
        pltpu.sync_copy(dst_hbm.at[pl.ds(off, C)], idxv)
        cp_u = pltpu.async_copy(h_hbm.at[idxu], urows, sem)
        cp_v = pltpu.async_copy(h_hbm.at[idxv], vrows, sem)
        cp_u.wait()
        cp_v.wait()

        def group_body(g, carry2):
            tot = jnp.zeros((L,), jnp.float32)
            for j in range(L):
                e = g * L + j
                acc = urows[e, pl.ds(0, L)] * vrows[e, pl.ds(0, L)]
                for c in range(1, D // L):
                    acc = acc + (urows[e, pl.ds(c * L, L)]
                                 * vrows[e, pl.ds(c * L, L)])
                s = acc[0]
                for l in range(1, L):
                    s = s + acc[l]
                tot = jnp.where(lanes == j, s, tot)
            outall[pl.ds(i * C + g * L, L)] = tot
            return carry2

        lax.fori_loop(0, C // L, group_body, 0)
        return carry

    lax.fori_loop(0, N_CHUNKS, chunk_body, 0)
    pltpu.sync_copy(outall, out_hbm.at[pl.ds(base, PER_TILE)])


def kernel(h, edge_index):
    edge_index = edge_index.astype(jnp.int32)
    src = edge_index[0]
    dst = edge_index[1]
    score = _edge_dot(src, dst, h)
    return score.reshape(E, 1)


# speedup vs baseline: 7.1983x; 2.1806x over previous
import functools

import jax
import jax.numpy as jnp
from jax import lax
from jax.experimental import pallas as pl
from jax.experimental.pallas import tpu as pltpu
from jax.experimental.pallas import tpu_sc as plsc

E = 320000
D = 128
L = 16
NC = 2
NS = 16
NW = NC * NS
PER_TILE = E // NW
C = 80
N_CHUNKS = PER_TILE // C


@functools.partial(
    pl.kernel,
    mesh=plsc.VectorSubcoreMesh(core_axis_name="c", subcore_axis_name="s"),
    out_type=jax.ShapeDtypeStruct((E,), jnp.float32),
    scratch_types=[
        pltpu.VMEM((PER_TILE,), jnp.int32),
        pltpu.VMEM((PER_TILE,), jnp.int32),
        pltpu.VMEM((2, C, D), jnp.float32),
        pltpu.VMEM((2, C, D), jnp.float32),
        pltpu.VMEM((PER_TILE,), jnp.float32),
        pltpu.SemaphoreType.DMA,
        pltpu.SemaphoreType.DMA,
    ],
)
def _edge_dot(src_hbm, dst_hbm, h_hbm, out_hbm,
              idxu, idxv, ubuf, vbuf, outall, sem0, sem1):
    wid = lax.axis_index("s") * NC + lax.axis_index("c")
    base = wid * PER_TILE
    lanes = lax.iota(jnp.int32, L)
    sems = (sem0, sem1)

    pltpu.sync_copy(src_hbm.at[pl.ds(base, PER_TILE)], idxu)
    pltpu.sync_copy(dst_hbm.at[pl.ds(base, PER_TILE)], idxv)

    def start_gather(chunk, buf):
        off = chunk * C
        cu = pltpu.async_copy(h_hbm.at[idxu.at[pl.ds(off, C)]],
                              ubuf.at[buf], sems[buf])
        cv = pltpu.async_copy(h_hbm.at[idxv.at[pl.ds(off, C)]],
                              vbuf.at[buf], sems[buf])
        return cu, cv

    def compute(chunk, buf, cu, cv):
        cu.wait()
        cv.wait()
        urows = ubuf.at[buf]
        vrows = vbuf.at[buf]

        def group_body(g, carry):
            tot = jnp.zeros((L,), jnp.float32)
            for j in range(L):
                acc = urows[g * L + j, pl.ds(0, L)] * vrows[g * L + j, pl.ds(0, L)]
                for c in range(1, D // L):
                    acc = acc + (urows[g * L + j, pl.ds(c * L, L)]
                                 * vrows[g * L + j, pl.ds(c * L, L)])
                s = acc[0]
                for l in range(1, L):
                    s = s + acc[l]
                tot = jnp.where(lanes == j, s, tot)
            outall[pl.ds(chunk * C + g * L, L)] = tot
            return carry

        lax.fori_loop(0, C // L, group_body, 0)

    cu0, cv0 = start_gather(0, 0)

    def pair_body(k, carry):
        c0 = 2 * k
        cu1, cv1 = start_gather(c0 + 1, 1)
        compute(c0, 0, cu0, cv0)
        cu0b, cv0b = start_gather(c0 + 2, 0)
        compute(c0 + 1, 1, cu1, cv1)
        return carry

    lax.fori_loop(0, (N_CHUNKS - 1) // 2, pair_body, 0)
    compute(N_CHUNKS - 1, 0, cu0, cv0)

    pltpu.sync_copy(outall, out_hbm.at[pl.ds(base, PER_TILE)])


def kernel(h, edge_index):
    edge_index = edge_index.astype(jnp.int32)
    src = edge_index[0]
    dst = edge_index[1]
    score = _edge_dot(src, dst, h)
    return score.reshape(E, 1)


# speedup vs baseline: 7.4977x; 1.0416x over previous
import functools

import jax
import jax.numpy as jnp
from jax import lax
from jax.experimental import pallas as pl
from jax.experimental.pallas import tpu as pltpu
from jax.experimental.pallas import tpu_sc as plsc

E = 320000
D = 128
L = 16
NC = 2
NS = 16
NW = NC * NS
PER_TILE = E // NW
C = 80
N_CHUNKS = PER_TILE // C


@functools.partial(
    pl.kernel,
    mesh=plsc.VectorSubcoreMesh(core_axis_name="c", subcore_axis_name="s"),
    out_type=jax.ShapeDtypeStruct((E,), jnp.float32),
    compiler_params=pltpu.CompilerParams(needs_layout_passes=False,
                                         use_tc_tiling_on_sc=False),
    scratch_types=[
        pltpu.VMEM((PER_TILE,), jnp.int32),
        pltpu.VMEM((PER_TILE,), jnp.int32),
        pltpu.VMEM((2, C, D // 2), jnp.int32),
        pltpu.VMEM((2, C, D // 2), jnp.int32),
        pltpu.VMEM((PER_TILE,), jnp.float32),
        pltpu.SemaphoreType.DMA,
        pltpu.SemaphoreType.DMA,
    ],
)
def _edge_dot(src_hbm, dst_hbm, h_hbm, out_hbm,
              idxu, idxv, ubuf, vbuf, outall, sem0, sem1):
    wid = lax.axis_index("s") * NC + lax.axis_index("c")
    base = wid * PER_TILE
    lanes = lax.iota(jnp.int32, L)
    sems = (sem0, sem1)

    def lane_shuffle(x, idx):
        return lax.gather(
            x, idx[:, None],
            dimension_numbers=lax.GatherDimensionNumbers(
                offset_dims=(), collapsed_slice_dims=(0,),
                start_index_map=(0,)),
            slice_sizes=(1,),
            mode=lax.GatherScatterMode.PROMISE_IN_BOUNDS)

    himask = jnp.full((L,), -65536, jnp.int32)

    def widen(pair_bits):
        lo = lax.bitcast_convert_type(
            lax.shift_left(pair_bits, 16), jnp.float32)
        hi = lax.bitcast_convert_type(
            lax.bitwise_and(pair_bits, himask), jnp.float32)
        return lo, hi

    pltpu.sync_copy(src_hbm.at[pl.ds(base, PER_TILE)], idxu)
    pltpu.sync_copy(dst_hbm.at[pl.ds(base, PER_TILE)], idxv)

    def start_gather(chunk, buf):
        off = chunk * C
        cu = pltpu.async_copy(h_hbm.at[idxu.at[pl.ds(off, C)]],
                              ubuf.at[buf], sems[buf])
        cv = pltpu.async_copy(h_hbm.at[idxv.at[pl.ds(off, C)]],
                              vbuf.at[buf], sems[buf])
        return cu, cv

    def compute(chunk, buf, cu, cv):
        cu.wait()
        cv.wait()
        urows = ubuf.at[buf]
        vrows = vbuf.at[buf]

        def group_body(g, carry):
            tot = jnp.zeros((L,), jnp.float32)
            for j in range(L):
                e = g * L + j
                acc = jnp.zeros((L,), jnp.float32)
                for c in range(D // (2 * L)):
                    ub = urows[e, pl.ds(c * L, L)]
                    vb = vrows[e, pl.ds(c * L, L)]
                    ulo, uhi = widen(ub)
                    vlo, vhi = widen(vb)
                    acc = acc + ulo * vlo + uhi * vhi
                for dist in (8, 4, 2, 1):
                    acc = acc + lane_shuffle(acc, lanes ^ dist)
                tot = jnp.where(lanes == j, acc, tot)
            outall[pl.ds(chunk * C + g * L, L)] = tot
            return carry

        lax.fori_loop(0, C // L, group_body, 0)

    cu0, cv0 = start_gather(0, 0)

    def pair_body(k, carry):
        c0 = 2 * k
        cu1, cv1 = start_gather(c0 + 1, 1)
        compute(c0, 0, cu0, cv0)
        start_gather(c0 + 2, 0)
        compute(c0 + 1, 1, cu1, cv1)
        return carry

    lax.fori_loop(0, (N_CHUNKS - 1) // 2, pair_body, 0)
    compute(N_CHUNKS - 1, 0, cu0, cv0)

    pltpu.sync_copy(outall, out_hbm.at[pl.ds(base, PER_TILE)])


def kernel(h, edge_index):
    edge_index = edge_index.astype(jnp.int32)
    src = edge_index[0]
    dst = edge_index[1]
    hb = h.astype(jnp.bfloat16).reshape(h.shape[0], h.shape[1] // 2, 2)
    h32 = lax.bitcast_convert_type(hb, jnp.int32)
    score = _edge_dot(src, dst, h32)
    return score.reshape(E, 1)


# speedup vs baseline: 9.0225x; 1.2034x over previous
import functools

import jax
import jax.numpy as jnp
from jax import lax
from jax.experimental import pallas as pl
from jax.experimental.pallas import tpu as pltpu
from jax.experimental.pallas import tpu_sc as plsc

E = 320000
D = 128
W = D // 2
L = 16
NC = 2
NS = 16
NW = NC * NS
PER_TILE = E // NW
C = 400
N_CHUNKS = PER_TILE // C


@functools.partial(
    pl.kernel,
    mesh=plsc.VectorSubcoreMesh(core_axis_name="c", subcore_axis_name="s"),
    out_type=jax.ShapeDtypeStruct((E,), jnp.float32),
    compiler_params=pltpu.CompilerParams(needs_layout_passes=False,
                                         use_tc_tiling_on_sc=False),
    scratch_types=[
        pltpu.VMEM((2, C), jnp.int32),
        pltpu.VMEM((2, C), jnp.int32),
        pltpu.VMEM((2, C, W), jnp.int32),
        pltpu.VMEM((2, C, W), jnp.int32),
        pltpu.VMEM((PER_TILE,), jnp.float32),
        pltpu.SemaphoreType.DMA,
        pltpu.SemaphoreType.DMA,
    ],
)
def _edge_dot(src_hbm, dst_hbm, h_hbm, out_hbm,
              idxu, idxv, ubuf, vbuf, outall, sem0, sem1):
    wid = lax.axis_index("s") * NC + lax.axis_index("c")
    base = wid * PER_TILE
    lanes = lax.iota(jnp.int32, L)
    sems = (sem0, sem1)
    himask = jnp.full((L,), -65536, jnp.int32)

    def lane_shuffle(x, idx):
        return lax.gather(
            x, idx[:, None],
            dimension_numbers=lax.GatherDimensionNumbers(
                offset_dims=(), collapsed_slice_dims=(0,),
                start_index_map=(0,)),
            slice_sizes=(1,),
            mode=lax.GatherScatterMode.PROMISE_IN_BOUNDS)

    def widen(pair_bits):
        lo = lax.bitcast_convert_type(
            lax.shift_left(pair_bits, 16), jnp.float32)
        hi = lax.bitcast_convert_type(
            lax.bitwise_and(pair_bits, himask), jnp.float32)
        return lo, hi

    def start_gather(chunk, buf):
        off = base + chunk * C
        pltpu.sync_copy(src_hbm.at[pl.ds(off, C)], idxu.at[buf])
        pltpu.sync_copy(dst_hbm.at[pl.ds(off, C)], idxv.at[buf])
        cu = pltpu.async_copy(h_hbm.at[idxu.at[buf]], ubuf.at[buf], sems[buf])
        cv = pltpu.async_copy(h_hbm.at[idxv.at[buf]], vbuf.at[buf], sems[buf])
        return cu, cv

    def compute(chunk, buf, cu, cv):
        cu.wait()
        cv.wait()
        urows = ubuf.at[buf]
        vrows = vbuf.at[buf]

        def group_body(g, carry):
            tot = jnp.zeros((L,), jnp.float32)
            for j in range(L):
                e = g * L + j
                acc = jnp.zeros((L,), jnp.float32)
                for c in range(W // L):
                    ub = urows[e, pl.ds(c * L, L)]
                    vb = vrows[e, pl.ds(c * L, L)]
                    ulo, uhi = widen(ub)
                    vlo, vhi = widen(vb)
                    acc = acc + ulo * vlo + uhi * vhi
                for dist in (8, 4, 2, 1):
                    acc = acc + lane_shuffle(acc, lanes ^ dist)
                tot = jnp.where(lanes == j, acc, tot)
            outall[pl.ds(chunk * C + g * L, L)] = tot
            return carry

        lax.fori_loop(0, C // L, group_body, 0)

    cu0, cv0 = start_gather(0, 0)

    def pair_body(k, carry):
        c0 = 2 * k
        cu1, cv1 = start_gather(c0 + 1, 1)
        compute(c0, 0, cu0, cv0)
        start_gather(c0 + 2, 0)
        compute(c0 + 1, 1, cu1, cv1)
        return carry

    lax.fori_loop(0, (N_CHUNKS - 1) // 2, pair_body, 0)
    compute(N_CHUNKS - 1, 0, cu0, cv0)

    pltpu.sync_copy(outall, out_hbm.at[pl.ds(base, PER_TILE)])


def kernel(h, edge_index):
    edge_index = edge_index.astype(jnp.int32)
    src = edge_index[0]
    dst = edge_index[1]
    hb = h.astype(jnp.bfloat16).reshape(h.shape[0], h.shape[1] // 2, 2)
    h32 = lax.bitcast_convert_type(hb, jnp.int32)
    score = _edge_dot(src, dst, h32)
    return score.reshape(E, 1)
